# baseline (speedup 1.0000x reference)
import functools

import jax
import jax.numpy as jnp
from jax import lax
from jax.experimental import pallas as pl
from jax.experimental.pallas import tpu as pltpu
from jax.experimental.pallas import tpu_sc as plsc

KEEP_RATIO = 0.5


def _sc_scores_body(x_hbm, out_hbm, buf0, buf1, scores_v, sem0, sem1):
    w = lax.axis_index("s") * 2 + lax.axis_index("c")
    nrow = x_hbm.shape[2]
    c = x_hbm.shape[1]
    lanes = 16

    tail = jnp.where(lax.iota(jnp.int32, lanes) >= 8, 1.0, 0.0)
    row_ids = lax.iota(jnp.int32, lanes)

    def image_sum(buf):
        acc = jnp.zeros((lanes,), jnp.float32)
        for r in range(nrow):
            v0 = jnp.abs(buf[r, 0:16])
            v1 = jnp.abs(buf[r, 16:32])
            v2 = jnp.abs(buf[r, 32:48])
            v3 = jnp.abs(buf[r, 40:56]) * tail
            acc = acc + (v0 + v1) + (v2 + v3)
        return acc

    pltpu.async_copy(x_hbm.at[w, 0], buf0, sem0)
    pltpu.async_copy(x_hbm.at[w, 1], buf1, sem1)

    def chan_pair(k, _):
        ch0 = 2 * k
        pltpu.make_async_copy(x_hbm.at[w, ch0], buf0, sem0).wait()
        acc0 = image_sum(buf0)

        @pl.when(ch0 + 2 < c)
        def _pf0():
            pltpu.async_copy(x_hbm.at[w, ch0 + 2], buf0, sem0)

        scores_v[pl.ds(ch0 * 16, 16)] = acc0

        ch1 = ch0 + 1
        pltpu.make_async_copy(x_hbm.at[w, ch1], buf1, sem1).wait()
        acc1 = image_sum(buf1)

        @pl.when(ch1 + 2 < c)
        def _pf1():
            pltpu.async_copy(x_hbm.at[w, ch1 + 2], buf1, sem1)

        scores_v[pl.ds(ch1 * 16, 16)] = acc1
        return 0

    lax.fori_loop(0, c // 2, chan_pair, 0)
    pltpu.sync_copy(scores_v, out_hbm.at[w])


def _mask_body(s3_ref, o_ref):
    b = s3_ref.shape[0]
    c = s3_ref.shape[1]
    k = max(1, min(c, int(round(c * KEEP_RATIO))))
    scores = jnp.sum(s3_ref[...], axis=2)
    bits = jax.lax.bitcast_convert_type(scores, jnp.int32)
    col = jax.lax.broadcasted_iota(jnp.int32, (b, c), 1)

    def vsearch(_, carry):
        lo, hi = carry
        mid = lo + ((hi - lo + 1) >> 1)
        cnt = jnp.sum((bits >= mid).astype(jnp.int32), axis=1, keepdims=True)
        p = cnt >= k
        return jnp.where(p, mid, lo), jnp.where(p, hi, mid - 1)

    lo = jnp.zeros((b, 1), jnp.int32)
    hi = jnp.full((b, 1), 0x7F800000, jnp.int32)
    t, _ = jax.lax.fori_loop(0, 31, vsearch, (lo, hi))

    gt = bits > t
    eq = bits == t
    need_eq = k - jnp.sum(gt.astype(jnp.int32), axis=1, keepdims=True)

    def isearch(_, carry):
        lo2, hi2 = carry
        mid = (lo2 + hi2) >> 1
        cnt = jnp.sum((eq & (col <= mid)).astype(jnp.int32), axis=1,
                      keepdims=True)
        p = cnt >= need_eq
        return jnp.where(p, lo2, mid + 1), jnp.where(p, mid, hi2)

    lo2 = jnp.zeros((b, 1), jnp.int32)
    hi2 = jnp.full((b, 1), c - 1, jnp.int32)
    m, _ = jax.lax.fori_loop(0, 10, isearch, (lo2, hi2))

    o_ref[...] = (gt | (eq & (col <= m))).astype(jnp.float32)


def kernel(features, enabled):
    b, c, h, w = features.shape

    sc_scores = functools.partial(
        pl.kernel,
        mesh=plsc.VectorSubcoreMesh(core_axis_name="c", subcore_axis_name="s"),
        out_type=jax.ShapeDtypeStruct((b, 16 * c), jnp.float32),
        scratch_types=[
            pltpu.VMEM((h, w), jnp.float32),
            pltpu.VMEM((h, w), jnp.float32),
            pltpu.VMEM((16 * c,), jnp.float32),
            pltpu.SemaphoreType.DMA,
            pltpu.SemaphoreType.DMA,
        ],
    )(_sc_scores_body)
    partials = sc_scores(features).reshape(b, c, 16)

    mask = pl.pallas_call(
        _mask_body,
        out_shape=jax.ShapeDtypeStruct((b, c), jnp.float32),
    )(partials)

    gated = mask.reshape(b, c, 1, 1)
    return jnp.where(jnp.asarray(enabled) != 0, gated,
                     jnp.ones_like(gated))


# speedup vs baseline: 1.2108x; 1.2108x over previous
import functools

import jax
import jax.numpy as jnp
from jax import lax
from jax.experimental import pallas as pl
from jax.experimental.pallas import tpu as pltpu
from jax.experimental.pallas import tpu_sc as plsc

KEEP_RATIO = 0.5


_CH = 4
_NBUF = 4


def _sc_scores_body(x_hbm, out_hbm, b0, b1, b2, b3, scores_v,
                    s0, s1, s2, s3):
    w = lax.axis_index("s") * 2 + lax.axis_index("c")
    c = x_hbm.shape[1]
    lanes = 16
    nchunk = c // _CH
    bufs = (b0, b1, b2, b3)
    sems = (s0, s1, s2, s3)

    tail = jnp.where(lax.iota(jnp.int32, lanes) >= 8, 1.0, 0.0)

    def image_sum(buf, j):
        def rows4(q, acc):
            for rr in range(4):
                r = 4 * q + rr
                acc = acc + (jnp.abs(buf[j, r, 0:16]) +
                             jnp.abs(buf[j, r, 16:32])) + \
                            (jnp.abs(buf[j, r, 32:48]) +
                             jnp.abs(buf[j, r, 40:56]) * tail)
            return acc
        return lax.fori_loop(0, 14, rows4, jnp.zeros((lanes,), jnp.float32))

    for t in range(_NBUF):
        pltpu.async_copy(x_hbm.at[w, pl.ds(t * _CH, _CH)], bufs[t], sems[t])

    def ring_step(k, _):
        for t in range(_NBUF):
            chunk = _NBUF * k + t
            cb = chunk * _CH
            pltpu.make_async_copy(x_hbm.at[w, pl.ds(cb, _CH)],
                                  bufs[t], sems[t]).wait()
            accs = [image_sum(bufs[t], j) for j in range(_CH)]

            @pl.when(chunk + _NBUF < nchunk)
            def _pf():
                pltpu.async_copy(
                    x_hbm.at[w, pl.ds(cb + _NBUF * _CH, _CH)],
                    bufs[t], sems[t])

            for j in range(_CH):
                scores_v[pl.ds((cb + j) * 16, 16)] = accs[j]
        return 0

    lax.fori_loop(0, nchunk // _NBUF, ring_step, 0)
    pltpu.sync_copy(scores_v, out_hbm.at[w])


def _mask_body(s3_ref, o_ref):
    b = s3_ref.shape[0]
    c = s3_ref.shape[1]
    k = max(1, min(c, int(round(c * KEEP_RATIO))))
    scores = jnp.sum(s3_ref[...], axis=2)
    bits = jax.lax.bitcast_convert_type(scores, jnp.int32)
    col = jax.lax.broadcasted_iota(jnp.int32, (b, c), 1)

    def vsearch(_, carry):
        lo, hi = carry
        mid = lo + ((hi - lo + 1) >> 1)
        cnt = jnp.sum((bits >= mid).astype(jnp.int32), axis=1, keepdims=True)
        p = cnt >= k
        return jnp.where(p, mid, lo), jnp.where(p, hi, mid - 1)

    lo = jnp.zeros((b, 1), jnp.int32)
    hi = jnp.full((b, 1), 0x7F800000, jnp.int32)
    t, _ = jax.lax.fori_loop(0, 31, vsearch, (lo, hi))

    gt = bits > t
    eq = bits == t
    need_eq = k - jnp.sum(gt.astype(jnp.int32), axis=1, keepdims=True)

    def isearch(_, carry):
        lo2, hi2 = carry
        mid = (lo2 + hi2) >> 1
        cnt = jnp.sum((eq & (col <= mid)).astype(jnp.int32), axis=1,
                      keepdims=True)
        p = cnt >= need_eq
        return jnp.where(p, lo2, mid + 1), jnp.where(p, mid, hi2)

    lo2 = jnp.zeros((b, 1), jnp.int32)
    hi2 = jnp.full((b, 1), c - 1, jnp.int32)
    m, _ = jax.lax.fori_loop(0, 10, isearch, (lo2, hi2))

    o_ref[...] = (gt | (eq & (col <= m))).astype(jnp.float32)


def kernel(features, enabled):
    b, c, h, w = features.shape

    sc_scores = functools.partial(
        pl.kernel,
        mesh=plsc.VectorSubcoreMesh(core_axis_name="c", subcore_axis_name="s"),
        out_type=jax.ShapeDtypeStruct((b, 16 * c), jnp.float32),
        scratch_types=(
            [pltpu.VMEM((_CH, h, w), jnp.float32) for _ in range(_NBUF)]
            + [pltpu.VMEM((16 * c,), jnp.float32)]
            + [pltpu.SemaphoreType.DMA for _ in range(_NBUF)]
        ),
    )(_sc_scores_body)
    partials = sc_scores(features).reshape(b, c, 16)

    mask = pl.pallas_call(
        _mask_body,
        out_shape=jax.ShapeDtypeStruct((b, c), jnp.float32),
    )(partials)

    gated = mask.reshape(b, c, 1, 1)
    return jnp.where(jnp.asarray(enabled) != 0, gated,
                     jnp.ones_like(gated))
